# Initial kernel scaffold; baseline (speedup 1.0000x reference)
import jax
import jax.numpy as jnp
from jax.experimental import pallas as pl


def kernel(feat, coord, relation, edge_weight_per_type_per_sent, fusion_edgefeat_per_node, Wr1, br1, Wr2, br2, Wf1, bf1, Wf2, bf2, Wg1, bg1, Wg2, bg2, gamma, beta):
    raise NotImplementedError("write your pallas kernel here")


# speedup vs baseline: 13.0620x; 13.0620x over previous
import functools

import jax
import jax.numpy as jnp
from jax.experimental import pallas as pl
from jax.experimental.pallas import tpu as pltpu


def _leaky(x):
    return jnp.where(x > 0, x, 0.01 * x)


def _body(n, k1,
          feat_ref, coord_ref, coordt_ref, rel_ref, ew_ref, fuse_ref,
          wr1a_ref, wr1b_ref, wr1c_ref, wr1d_ref, br1_ref, wr2_ref, br2_ref,
          wf1_ref, bf1_ref, wf2_ref, bf2_ref,
          wg1_ref, bg1_ref, wg2_ref, bg2_ref,
          gamma_ref, beta_ref, out_ref, fe_ref, ef_ref):
    f32 = jnp.float32
    coord8 = coord_ref[0]

    fe = _leaky(jnp.dot(feat_ref[0], wf1_ref[:], preferred_element_type=f32)
                + bf1_ref[:])
    fe_ref[:] = jnp.dot(fe, wf2_ref[:], preferred_element_type=f32) + bf2_ref[:]
    ef = _leaky(jnp.dot(fuse_ref[0], wg1_ref[:], preferred_element_type=f32)
                + bg1_ref[:])
    ef_ref[:] = jnp.dot(ef, wg2_ref[:], preferred_element_type=f32) + bg2_ref[:]

    d = jnp.zeros((n, n), f32)
    for c in range(3):
        ucol = coord8[:, c:c + 1]
        urow = coordt_ref[0, c:c + 1, :]
        diff = urow - ucol
        d = d + diff * diff

    relation = rel_ref[:]
    wm = jnp.zeros((n, n), f32)
    for t in range(16):
        wm = wm + jnp.where(relation == t, ew_ref[0, 0, t], 0.0)

    base = (jnp.dot(coord8, wr1b_ref[:], preferred_element_type=f32)
            + br1_ref[:])

    colidx = jax.lax.broadcasted_iota(jnp.int32, (n, n), 1)

    def round_body(_, carry):
        d, acc, wadj = carry
        rowmin = jnp.min(d, axis=1, keepdims=True)
        ismin = d <= rowmin
        idx = jnp.min(jnp.where(ismin, colidx, n), axis=1, keepdims=True)
        hit = colidx == idx
        oh = hit.astype(f32)
        d = jnp.where(hit, jnp.inf, d)
        wadj = wadj + oh * wm

        g = jnp.dot(oh, fe_ref[:], preferred_element_type=f32)
        ce = jnp.dot(oh, coord8, preferred_element_type=f32)
        rc = ce - coord8
        dis = jnp.sqrt(jnp.sum(rc * rc, axis=1, keepdims=True))
        h = (jnp.dot(ce, wr1a_ref[:], preferred_element_type=f32)
             + jnp.dot(rc, wr1c_ref[:], preferred_element_type=f32)
             + dis * wr1d_ref[:]
             + base)
        r = jnp.dot(_leaky(h), wr2_ref[:], preferred_element_type=f32) + br2_ref[:]
        acc = acc + g * r
        return d, acc, wadj

    acc0 = jnp.zeros((n, out_ref.shape[-1]), f32)
    wadj0 = jnp.zeros((n, n), f32)
    d, acc, wadj = jax.lax.fori_loop(0, k1, round_body, (d, acc0, wadj0))

    term2 = jnp.dot(wadj, ef_ref[:], preferred_element_type=f32)
    residual = acc + term2 + fe_ref[:] + ef_ref[:]

    mu = jnp.mean(residual, axis=1, keepdims=True)
    var = jnp.mean((residual - mu) ** 2, axis=1, keepdims=True)
    out = ((residual - mu) / jnp.sqrt(var + 1e-5)) * gamma_ref[:] + beta_ref[:]
    out_ref[0] = out


def _full(shape):
    nd = len(shape)
    return pl.BlockSpec(shape, lambda s, _nd=nd: (0,) * _nd)


@jax.jit
def kernel(feat, coord, relation, edge_weight_per_type_per_sent,
           fusion_edgefeat_per_node,
           Wr1, br1, Wr2, br2, Wf1, bf1, Wf2, bf2, Wg1, bg1, Wg2, bg2,
           gamma, beta):
    s, n, din = feat.shape
    dout = Wf2.shape[0]
    k1 = min(n - 1, 16) + 1
    f32 = jnp.float32

    coord8 = jnp.pad(coord.astype(f32), ((0, 0), (0, 0), (0, 8 - coord.shape[-1])))
    relation = relation.astype(jnp.int32)
    ew3 = edge_weight_per_type_per_sent.astype(f32).reshape(s, 1, -1)

    wr1t = Wr1.T.astype(f32)
    pad3 = lambda w: jnp.pad(w, ((0, 8 - w.shape[0]), (0, 0)))
    wr1a = pad3(wr1t[0:3])
    wr1b = pad3(wr1t[3:6])
    wr1c = pad3(wr1t[6:9])
    wr1d = wr1t[9:10]
    row = lambda b: b.astype(f32).reshape(1, -1)

    coordt = jnp.swapaxes(coord8, 1, 2)
    operands = (
        feat.astype(f32), coord8, coordt, relation, ew3,
        fusion_edgefeat_per_node.astype(f32),
        wr1a, wr1b, wr1c, wr1d, row(br1), Wr2.T.astype(f32), row(br2),
        Wf1.T.astype(f32), row(bf1), Wf2.T.astype(f32), row(bf2),
        Wg1.T.astype(f32), row(bg1), Wg2.T.astype(f32), row(bg2),
        row(gamma), row(beta),
    )

    in_specs = [
        pl.BlockSpec((1, n, din), lambda s_: (s_, 0, 0)),
        pl.BlockSpec((1, n, 8), lambda s_: (s_, 0, 0)),
        pl.BlockSpec((1, 8, n), lambda s_: (s_, 0, 0)),
        _full(relation.shape),
        pl.BlockSpec((1, 1, ew3.shape[-1]), lambda s_: (s_, 0, 0)),
        pl.BlockSpec((1, n, din), lambda s_: (s_, 0, 0)),
    ] + [_full(op.shape) for op in operands[6:]]

    return pl.pallas_call(
        functools.partial(_body, n, k1),
        grid=(s,),
        in_specs=in_specs,
        out_specs=pl.BlockSpec((1, n, dout), lambda s_: (s_, 0, 0)),
        out_shape=jax.ShapeDtypeStruct((s, n, dout), f32),
        scratch_shapes=[
            pltpu.VMEM((n, dout), f32),
            pltpu.VMEM((n, dout), f32),
        ],
        compiler_params=pltpu.CompilerParams(
            dimension_semantics=("arbitrary",)),
    )(*operands)


# speedup vs baseline: 19.5584x; 1.4973x over previous
import functools

import jax
import jax.numpy as jnp
from jax.experimental import pallas as pl
from jax.experimental.pallas import tpu as pltpu


def _leaky(x):
    return jnp.where(x > 0, x, 0.01 * x)


def _body(n, k1,
          feat_ref, coord_ref, coordt_ref, rel_ref, ew_ref, fuse_ref,
          wr1ac_ref, wr1bc_ref, wr1d_ref, br1_ref, wr2_ref, br2_ref,
          wf1_ref, bf1_ref, wf2_ref, bf2_ref,
          wg1_ref, bg1_ref, wg2_ref, bg2_ref,
          gamma_ref, beta_ref, out_ref, t_ref, ef_ref):
    f32 = jnp.float32
    dout = out_ref.shape[-1]
    coord8 = coord_ref[0]

    fe = _leaky(jnp.dot(feat_ref[0], wf1_ref[:], preferred_element_type=f32)
                + bf1_ref[:])
    t_ref[:, :dout] = (jnp.dot(fe, wf2_ref[:], preferred_element_type=f32)
                       + bf2_ref[:])
    t_ref[:, dout:] = jnp.dot(coord8, wr1ac_ref[:], preferred_element_type=f32)
    ef = _leaky(jnp.dot(fuse_ref[0], wg1_ref[:], preferred_element_type=f32)
                + bg1_ref[:])
    ef_ref[:] = jnp.dot(ef, wg2_ref[:], preferred_element_type=f32) + bg2_ref[:]

    d = jnp.zeros((n, n), f32)
    for c in range(3):
        ucol = coord8[:, c:c + 1]
        urow = coordt_ref[0, c:c + 1, :]
        diff = urow - ucol
        d = d + diff * diff

    relation = rel_ref[:]
    wm = jnp.zeros((n, n), f32)
    for t in range(16):
        wm = wm + jnp.where(relation == t, ew_ref[0, 0, t], 0.0)

    base = (jnp.dot(coord8, wr1bc_ref[:], preferred_element_type=f32)
            + br1_ref[:])

    colf = jax.lax.broadcasted_iota(jnp.int32, (n, n), 1).astype(f32)

    def mxu_stage(oh, dterm, acc):
        u = jnp.dot(oh, t_ref[:], preferred_element_type=f32)
        h = u[:, dout:] + dterm + base
        r = jnp.dot(_leaky(h), wr2_ref[:], preferred_element_type=f32) + br2_ref[:]
        return acc + u[:, :dout] * r

    hw = wr1d_ref.shape[-1]

    def round_body(_, carry):
        d, acc, sel, oh_p, dt_p = carry
        rowmin = jnp.min(d, axis=1, keepdims=True)
        idx = jnp.min(jnp.where(d <= rowmin, colf, float(n)),
                      axis=1, keepdims=True)
        hit = colf == idx
        oh = hit.astype(f32)
        d = jnp.where(hit, jnp.inf, d)
        sel = sel + oh
        dterm = jnp.sqrt(jnp.broadcast_to(rowmin, (n, hw))) * wr1d_ref[:]
        acc = mxu_stage(oh_p, dt_p, acc)
        return d, acc, sel, oh, dterm

    acc0 = jnp.zeros((n, dout), f32)
    sel0 = jnp.zeros((n, n), f32)
    zrow = jnp.zeros((n, hw), f32)
    d, acc, sel, oh_l, dt_l = jax.lax.fori_loop(
        0, k1, round_body, (d, acc0, sel0, sel0, zrow))
    acc = mxu_stage(oh_l, dt_l, acc)

    term2 = jnp.dot(sel * wm, ef_ref[:], preferred_element_type=f32)
    residual = acc + term2 + t_ref[:, :dout] + ef_ref[:]

    mu = jnp.mean(residual, axis=1, keepdims=True)
    var = jnp.mean((residual - mu) ** 2, axis=1, keepdims=True)
    out = ((residual - mu) / jnp.sqrt(var + 1e-5)) * gamma_ref[:] + beta_ref[:]
    out_ref[0] = out


def _full(shape):
    nd = len(shape)
    return pl.BlockSpec(shape, lambda s, _nd=nd: (0,) * _nd)


@jax.jit
def kernel(feat, coord, relation, edge_weight_per_type_per_sent,
           fusion_edgefeat_per_node,
           Wr1, br1, Wr2, br2, Wf1, bf1, Wf2, bf2, Wg1, bg1, Wg2, bg2,
           gamma, beta):
    s, n, din = feat.shape
    dout = Wf2.shape[0]
    k1 = min(n - 1, 16) + 1
    f32 = jnp.float32

    coord8 = jnp.pad(coord.astype(f32), ((0, 0), (0, 0), (0, 8 - coord.shape[-1])))
    relation = relation.astype(jnp.int32)
    ew3 = edge_weight_per_type_per_sent.astype(f32).reshape(s, 1, -1)

    wr1t = Wr1.T.astype(f32)
    pad3 = lambda w: jnp.pad(w, ((0, 8 - w.shape[0]), (0, 0)))
    wr1ac = pad3(wr1t[0:3] + wr1t[6:9])
    wr1bc = pad3(wr1t[3:6] - wr1t[6:9])
    wr1d = wr1t[9:10]
    row = lambda b: b.astype(f32).reshape(1, -1)

    coordt = jnp.swapaxes(coord8, 1, 2)
    operands = (
        feat.astype(f32), coord8, coordt, relation, ew3,
        fusion_edgefeat_per_node.astype(f32),
        wr1ac, wr1bc, wr1d, row(br1), Wr2.T.astype(f32), row(br2),
        Wf1.T.astype(f32), row(bf1), Wf2.T.astype(f32), row(bf2),
        Wg1.T.astype(f32), row(bg1), Wg2.T.astype(f32), row(bg2),
        row(gamma), row(beta),
    )

    in_specs = [
        pl.BlockSpec((1, n, din), lambda s_: (s_, 0, 0)),
        pl.BlockSpec((1, n, 8), lambda s_: (s_, 0, 0)),
        pl.BlockSpec((1, 8, n), lambda s_: (s_, 0, 0)),
        _full(relation.shape),
        pl.BlockSpec((1, 1, ew3.shape[-1]), lambda s_: (s_, 0, 0)),
        pl.BlockSpec((1, n, din), lambda s_: (s_, 0, 0)),
    ] + [_full(op.shape) for op in operands[6:]]

    return pl.pallas_call(
        functools.partial(_body, n, k1),
        grid=(s,),
        in_specs=in_specs,
        out_specs=pl.BlockSpec((1, n, dout), lambda s_: (s_, 0, 0)),
        out_shape=jax.ShapeDtypeStruct((s, n, dout), f32),
        scratch_shapes=[
            pltpu.VMEM((n, dout + Wr2.shape[1]), f32),
            pltpu.VMEM((n, dout), f32),
        ],
        compiler_params=pltpu.CompilerParams(
            dimension_semantics=("parallel",)),
    )(*operands)


# speedup vs baseline: 26.4694x; 1.3534x over previous
import functools

import jax
import jax.numpy as jnp
from jax.experimental import pallas as pl
from jax.experimental.pallas import tpu as pltpu


def _leaky(x):
    return jnp.where(x > 0, x, 0.01 * x)


def _body(n, k1,
          feat_ref, coord_ref, coordt_ref, rel_ref, ew_ref, fuse_ref,
          wr1ac_ref, wr1bc_ref, wr1d_ref, br1_ref, wr2_ref, br2_ref,
          wf1_ref, bf1_ref, wf2_ref, bf2_ref,
          wg1_ref, bg1_ref, wg2_ref, bg2_ref,
          gamma_ref, beta_ref, out_ref, t_ref, ef_ref):
    f32 = jnp.float32
    dout = out_ref.shape[-1]
    coord8 = coord_ref[0]

    fe = _leaky(jnp.dot(feat_ref[0], wf1_ref[:], preferred_element_type=f32)
                + bf1_ref[:])
    t_ref[:, :dout] = (jnp.dot(fe, wf2_ref[:], preferred_element_type=f32)
                       + bf2_ref[:])
    t_ref[:, dout:] = jnp.dot(coord8, wr1ac_ref[:], preferred_element_type=f32)

    ef = _leaky(jnp.dot(fuse_ref[0], wg1_ref[:], preferred_element_type=f32)
                + bg1_ref[:])
    ef_ref[:] = jnp.dot(ef, wg2_ref[:], preferred_element_type=f32) + bg2_ref[:]

    d = jnp.zeros((n, n), f32)
    for c in range(3):
        ucol = coord8[:, c:c + 1]
        urow = coordt_ref[0, c:c + 1, :]
        diff = urow - ucol
        d = d + diff * diff


    base = (jnp.dot(coord8, wr1bc_ref[:], preferred_element_type=f32)
            + br1_ref[:])

    colf = jax.lax.broadcasted_iota(jnp.int32, (n, n), 1).astype(f32)
    rowf = jax.lax.broadcasted_iota(jnp.int32, (n, n), 0).astype(f32)
    eyehit = colf == rowf
    eye = eyehit.astype(f32)
    d = jnp.where(eyehit, jnp.inf, d)

    hw = wr1d_ref.shape[-1]

    def mxu_stage(oh, dterm, acc):
        u = jnp.dot(oh, t_ref[:], preferred_element_type=f32)
        h = u[:, dout:] + dterm + base
        r = jnp.dot(_leaky(h), wr2_ref[:], preferred_element_type=f32) + br2_ref[:]
        return acc + u[:, :dout] * r

    def one_round(carry):
        d, acc, sel, oh_p, dt_p = carry
        rowmin = jnp.min(d, axis=1, keepdims=True)
        idx = jnp.min(jnp.where(d <= rowmin, colf, float(n)),
                      axis=1, keepdims=True)
        hit = colf == idx
        oh = hit.astype(f32)
        d = jnp.where(hit, jnp.inf, d)
        sel = sel + oh
        dterm = jnp.sqrt(jnp.broadcast_to(rowmin, (n, hw))) * wr1d_ref[:]
        acc = mxu_stage(oh_p, dt_p, acc)
        return d, acc, sel, oh, dterm

    def round_body(_, carry):
        return one_round(one_round(carry))

    acc0 = ef_ref[:] * 0.0
    zrow = base * 0.0
    d, acc, sel, oh_l, dt_l = jax.lax.fori_loop(
        0, (k1 - 1) // 2, round_body, (d, acc0, eye, eye, zrow))
    for _ in range((k1 - 1) % 2):
        d, acc, sel, oh_l, dt_l = one_round((d, acc, sel, oh_l, dt_l))
    acc = mxu_stage(oh_l, dt_l, acc)

    relation = rel_ref[:]
    bits = [(relation & (1 << b)) != 0 for b in range(4)]

    def lut(bit, lo):
        if bit == 0:
            return jnp.full((n, n), ew_ref[0, 0, lo], f32)
        half = 1 << (bit - 1)
        return jnp.where(bits[bit - 1], lut(bit - 1, lo + half), lut(bit - 1, lo))

    wm = lut(4, 0)

    term2 = jnp.dot(sel * wm, ef_ref[:], preferred_element_type=f32)
    residual = acc + term2 + t_ref[:, :dout] + ef_ref[:]

    mu = jnp.mean(residual, axis=1, keepdims=True)
    var = jnp.mean((residual - mu) ** 2, axis=1, keepdims=True)
    out = ((residual - mu) / jnp.sqrt(var + 1e-5)) * gamma_ref[:] + beta_ref[:]
    out_ref[0] = out


def _full(shape):
    nd = len(shape)
    return pl.BlockSpec(shape, lambda s, _nd=nd: (0,) * _nd)


@jax.jit
def kernel(feat, coord, relation, edge_weight_per_type_per_sent,
           fusion_edgefeat_per_node,
           Wr1, br1, Wr2, br2, Wf1, bf1, Wf2, bf2, Wg1, bg1, Wg2, bg2,
           gamma, beta):
    s, n, din = feat.shape
    dout = Wf2.shape[0]
    k1 = min(n - 1, 16) + 1
    f32 = jnp.float32

    coord8 = jnp.pad(coord.astype(f32), ((0, 0), (0, 0), (0, 8 - coord.shape[-1])))
    relation = relation.astype(jnp.int32)
    ew3 = edge_weight_per_type_per_sent.astype(f32).reshape(s, 1, -1)

    wr1t = Wr1.T.astype(f32)
    pad3 = lambda w: jnp.pad(w, ((0, 8 - w.shape[0]), (0, 0)))
    wr1ac = pad3(wr1t[0:3] + wr1t[6:9])
    wr1bc = pad3(wr1t[3:6] - wr1t[6:9])
    wr1d = wr1t[9:10]
    row = lambda b: b.astype(f32).reshape(1, -1)

    coordt = jnp.swapaxes(coord8, 1, 2)
    operands = (
        feat.astype(f32), coord8, coordt, relation, ew3,
        fusion_edgefeat_per_node.astype(f32),
        wr1ac, wr1bc, wr1d, row(br1), Wr2.T.astype(f32), row(br2),
        Wf1.T.astype(f32), row(bf1), Wf2.T.astype(f32), row(bf2),
        Wg1.T.astype(f32), row(bg1), Wg2.T.astype(f32), row(bg2),
        row(gamma), row(beta),
    )

    in_specs = [
        pl.BlockSpec((1, n, din), lambda s_: (s_, 0, 0)),
        pl.BlockSpec((1, n, 8), lambda s_: (s_, 0, 0)),
        pl.BlockSpec((1, 8, n), lambda s_: (s_, 0, 0)),
        _full(relation.shape),
        pl.BlockSpec((1, 1, ew3.shape[-1]), lambda s_: (s_, 0, 0)),
        pl.BlockSpec((1, n, din), lambda s_: (s_, 0, 0)),
    ] + [_full(op.shape) for op in operands[6:]]

    return pl.pallas_call(
        functools.partial(_body, n, k1),
        grid=(s,),
        in_specs=in_specs,
        out_specs=pl.BlockSpec((1, n, dout), lambda s_: (s_, 0, 0)),
        out_shape=jax.ShapeDtypeStruct((s, n, dout), f32),
        scratch_shapes=[
            pltpu.VMEM((n, dout + Wr2.shape[1]), f32),
            pltpu.VMEM((n, dout), f32),
        ],
        compiler_params=pltpu.CompilerParams(
            dimension_semantics=("parallel",)),
    )(*operands)
